# Initial kernel scaffold; baseline (speedup 1.0000x reference)
import jax
import jax.numpy as jnp
from jax.experimental import pallas as pl


def kernel(x, edge_index, edge_attr, batch, params):
    raise NotImplementedError("write your pallas kernel here")


# speedup vs baseline: 55.8591x; 55.8591x over previous
import functools

import jax
import jax.numpy as jnp
import numpy as np
from jax import lax
from jax.experimental import pallas as pl
from jax.experimental.pallas import tpu as pltpu
from jax.experimental.pallas import tpu_sc as plsc

_DEG = [1, 72, 201, 816, 1790, 3756, 6923, 12768, 20286, 31710, 51623,
        82296, 124280, 177576, 251115, 326064, 395760, 456840, 506179,
        516200, 507003, 493746, 489256, 453936, 420025, 411320, 427761,
        420700, 420500, 426780, 414284, 407008, 394053, 360910, 322245,
        313704, 282902, 270940, 237783, 209000, 193766, 177870, 162110,
        144848, 121230, 112700, 93483, 88512, 72275, 80700, 68799, 56784,
        42665, 30996, 25630, 12936, 9804, 8584, 5251, 3480, 3111, 2728,
        1890, 1472, 1235, 330, 201, 68, 69, 0, 71]
_dega = np.asarray(_DEG, dtype=np.float64)
_bins = np.arange(len(_DEG), dtype=np.float64)
_AVG_DEG_LOG = float((np.log(_bins + 1.0) * _dega).sum() / _dega.sum())

_N = 10000
_E = 160000
_DIM = 64
_T = 4
_F = 64
_FO = 16
_NG = 256
_EPS_BN = 1e-5

_NB = 128
_NPB = 80
_NPAD = _NB * _NPB
_K = 32
_EP = 160256
_BIG = 3.0e38


def _mm(a, b):
    return jnp.dot(a, b, preferred_element_type=jnp.float32,
                   precision=lax.Precision.HIGHEST)


def _embpre_body(x_ref, we_ref, be_ref, w1_ref, b1_ref, h_ref, xi_ref, xj_ref):
    h = _mm(x_ref[...], we_ref[...]) + be_ref[...]
    h_ref[...] = h
    xixj = _mm(h, w1_ref[...]) + b1_ref[...]
    xi_ref[...] = xixj[:, :256]
    xj_ref[...] = xixj[:, 256:]


def _prebn_body(o_ref, bn_ref, g_ref, b_ref, w1_ref, b1_ref,
                h_ref, xi_ref, xj_ref):
    s = bn_ref[...]
    m = s[0:1, :] * (1.0 / _N)
    var = s[1:2, :] * (1.0 / _N) - m * m
    o = o_ref[...]
    h = jax.nn.relu(g_ref[...] * (o - m) * lax.rsqrt(var + _EPS_BN) + b_ref[...])
    h_ref[...] = h
    xixj = _mm(h, w1_ref[...]) + b1_ref[...]
    xi_ref[...] = xixj[:, :256]
    xj_ref[...] = xixj[:, 256:]


def _edata_body(ea_ref, k_ref, c_ref, out_ref):
    out_ref[...] = _mm(ea_ref[...], k_ref[...]) + c_ref[...]


def _post_body(s1_ref, s2_ref, mn_ref, mx_ref, cnt_ref, xi_ref, h_ref,
               wabc_ref, wx_ref, bias_ref, out_ref, bn_ref):
    i = pl.program_id(0)
    c = cnt_ref[...][:, 0:1]
    has = c > 0.0
    cc = jnp.maximum(c, 1.0)
    inv = 1.0 / cc
    xi = xi_ref[...]
    s1 = s1_ref[...]
    e1 = s1 * inv
    mean = jnp.where(has, xi + e1, 0.0)
    mn = jnp.where(has, xi + mn_ref[...], 0.0)
    mx = jnp.where(has, xi + mx_ref[...], 0.0)
    var = s2_ref[...] * inv - e1 * e1
    std = jnp.sqrt(jax.nn.relu(var) + 1e-5)
    stat = jnp.concatenate([mean, mn, mx, std], axis=1)
    u = _mm(stat, wabc_ref[...])
    alpha = jnp.log(cc + 1.0) * (1.0 / _AVG_DEG_LOG)
    beta = _AVG_DEG_LOG / jnp.log(cc + 1.0)
    out = (_mm(h_ref[...], wx_ref[...]) + u[:, :64] + alpha * u[:, 64:128]
           + beta * u[:, 128:] + bias_ref[...])
    out_ref[...] = out

    @pl.when(i == 0)
    def _():
        bn_ref[...] = jnp.zeros_like(bn_ref)

    part = jnp.concatenate([jnp.sum(out, axis=0, keepdims=True),
                            jnp.sum(out * out, axis=0, keepdims=True)], axis=0)
    bn_ref[...] += part


def _final_body(o_ref, bn_ref, g_ref, b_ref, batch_ref, w1_ref, b1_ref,
                w2_ref, b2_ref, w3_ref, b3_ref, h_ref, gout_ref):
    s = bn_ref[...]
    m = s[0:1, :] * (1.0 / _N)
    var = s[1:2, :] * (1.0 / _N) - m * m
    h = jax.nn.relu(g_ref[...] * (o_ref[...] - m) * lax.rsqrt(var + _EPS_BN)
                    + b_ref[...])
    h_ref[...] = h
    onehot = (batch_ref[...] == lax.broadcasted_iota(jnp.int32, (1, _NG), 1)
              ).astype(jnp.float32)
    g = lax.dot_general(onehot, h, (((0,), (0,)), ((), ())),
                        preferred_element_type=jnp.float32,
                        precision=lax.Precision.HIGHEST)
    g = jax.nn.relu(_mm(g, w1_ref[...]) + b1_ref[...])
    g = jax.nn.relu(_mm(g, w2_ref[...]) + b2_ref[...])
    gout_ref[...] = _mm(g, w3_ref[...]) + b3_ref[...]


_ROWB = 2000


def _full(shape):
    return pl.BlockSpec(shape, lambda *a: tuple(0 for _ in shape))


def _rows(cols):
    return pl.BlockSpec((_ROWB, cols), lambda i: (i, 0))


def _embpre_call(x, we, be, w1, b1):
    return pl.pallas_call(
        _embpre_body,
        grid=(_N // _ROWB,),
        in_specs=[_rows(14), _full((14, 64)), _full((1, 64)),
                  _full((64, 512)), _full((1, 512))],
        out_specs=[_rows(64), _rows(256), _rows(256)],
        out_shape=[jax.ShapeDtypeStruct((_N, 64), jnp.float32),
                   jax.ShapeDtypeStruct((_N, 256), jnp.float32),
                   jax.ShapeDtypeStruct((_N, 256), jnp.float32)],
    )(x, we, be, w1, b1)


def _prebn_call(o, bn, g, b, w1, b1):
    return pl.pallas_call(
        _prebn_body,
        grid=(_N // _ROWB,),
        in_specs=[_rows(64), _full((2, 64)), _full((1, 64)), _full((1, 64)),
                  _full((64, 512)), _full((1, 512))],
        out_specs=[_rows(64), _rows(256), _rows(256)],
        out_shape=[jax.ShapeDtypeStruct((_N, 64), jnp.float32),
                   jax.ShapeDtypeStruct((_N, 256), jnp.float32),
                   jax.ShapeDtypeStruct((_N, 256), jnp.float32)],
    )(o, bn, g, b, w1, b1)


def _edata_call(ea8, kmat, cvec):
    return pl.pallas_call(
        _edata_body,
        grid=(_EP // 2504,),
        in_specs=[pl.BlockSpec((2504, 8), lambda i: (i, 0)),
                  _full((8, 256)), _full((1, 256))],
        out_specs=pl.BlockSpec((2504, 256), lambda i: (i, 0)),
        out_shape=jax.ShapeDtypeStruct((_EP, 256), jnp.float32),
    )(ea8, kmat, cvec)


def _post_call(s1, s2, mn, mx, cnt, xi, h, wabc, wx, bias):
    return pl.pallas_call(
        _post_body,
        grid=(_N // _ROWB,),
        in_specs=[_rows(256), _rows(256), _rows(256), _rows(256),
                  _rows(16), _rows(256), _rows(64),
                  _full((1024, 192)), _full((64, 64)), _full((1, 64))],
        out_specs=[_rows(64), _full((2, 64))],
        out_shape=[jax.ShapeDtypeStruct((_N, 64), jnp.float32),
                   jax.ShapeDtypeStruct((2, 64), jnp.float32)],
    )(s1, s2, mn, mx, cnt, xi, h, wabc, wx, bias)


def _final_call(o, bn, g, b, batch2d, w1, b1, w2, b2, w3, b3):
    return pl.pallas_call(
        _final_body,
        in_specs=[_full((_N, 64)), _full((2, 64)), _full((1, 64)),
                  _full((1, 64)), _full((_N, 1)), _full((64, 50)),
                  _full((1, 50)), _full((50, 25)), _full((1, 25)),
                  _full((25, 1)), _full((1, 1))],
        out_specs=[_full((_N, 64)), _full((_NG, 1))],
        out_shape=[jax.ShapeDtypeStruct((_N, 64), jnp.float32),
                   jax.ShapeDtypeStruct((_NG, 1), jnp.float32)],
    )(o, bn, g, b, batch2d, w1, b1, w2, b2, w3, b3)


_NC = 2
_NS = 16
_NW = _NC * _NS
_BPW = _NB // _NW


def _sc_read_scalar(vec_ref, idx):
    lanes = jnp.full((16,), idx, dtype=jnp.int32)
    vals = plsc.load_gather(vec_ref, [lanes])
    return jnp.max(vals, axis=0)


def _extract_lane_i32(vec, j):
    lane = lax.iota(jnp.int32, 16)
    return jnp.max(jnp.where(lane == j, vec, -1), axis=0)


def _sc_stats_body(xj_hbm, ed_hbm, src_hbm, dst_hbm, starts_hbm,
                   s1_hbm, s2_hbm, mn_hbm, mx_hbm, cnt_hbm,
                   starts_v, src_v, dst_v, ed_v, xj_v,
                   s1_v, s2_v, mn_v, mx_v, cnt_v,
                   sem2, sem3):
    wid = lax.axis_index("s") * _NC + lax.axis_index("c")
    pltpu.sync_copy(starts_hbm, starts_v)
    iota = lax.iota(jnp.int32, 16)
    zf = jnp.zeros((16,), jnp.float32)
    big = jnp.full((16,), _BIG, jnp.float32)
    onehot0 = jnp.where(iota == 0, 1.0, 0.0).astype(jnp.float32)
    for k in range(_BPW):
        b = wid * _BPW + k
        start = _sc_read_scalar(starts_v, b)
        end = _sc_read_scalar(starts_v, b + 1)
        node_base = b * _NPB

        def zrow(i, _):
            for c in range(16):
                s1_v[i, pl.ds(c * 16, 16)] = zf
                s2_v[i, pl.ds(c * 16, 16)] = zf
                mn_v[i, pl.ds(c * 16, 16)] = big
                mx_v[i, pl.ds(c * 16, 16)] = -big
            cnt_v[i, :] = zf
            return 0

        lax.fori_loop(0, _NPB + 1, zrow, 0)

        astart = (start // 8) * 8
        nch = (end - astart + _K - 1) // _K

        def chunk_body(i, _):
            base = astart + i * _K
            pltpu.sync_copy(src_hbm.at[pl.ds(base, _K)], src_v)
            pltpu.sync_copy(dst_hbm.at[pl.ds(base, _K)], dst_v)
            cp2 = pltpu.async_copy(ed_hbm.at[pl.ds(base, _K)], ed_v, sem2)
            cp3 = pltpu.async_copy(xj_hbm.at[src_v], xj_v, sem3)
            cp2.wait()
            cp3.wait()
            for g in range(_K // 16):
                dv = dst_v[pl.ds(g * 16, 16)]
                gidx = base + g * 16 + iota
                valid = (gidx >= start) & (gidx < end)
                dstloc = jnp.where(valid, dv - node_base, _NPB)

                def edge_body(j, _):
                    row = _extract_lane_i32(dstloc, j)
                    er = g * 16 + j
                    plsc.addupdate(cnt_v.at[row, :], onehot0)
                    for c in range(16):
                        xc = xj_v[er, pl.ds(c * 16, 16)]
                        ec = ed_v[er, pl.ds(c * 16, 16)]
                        y = xc + ec
                        plsc.addupdate(s1_v.at[row, pl.ds(c * 16, 16)], y)
                        plsc.addupdate(s2_v.at[row, pl.ds(c * 16, 16)], y * y)
                        cur = mn_v[row, pl.ds(c * 16, 16)]
                        mn_v[row, pl.ds(c * 16, 16)] = jnp.minimum(cur, y)
                        cur = mx_v[row, pl.ds(c * 16, 16)]
                        mx_v[row, pl.ds(c * 16, 16)] = jnp.maximum(cur, y)
                    return 0

                lax.fori_loop(0, 16, edge_body, 0)
            return 0

        lax.fori_loop(0, nch, chunk_body, 0)

        pltpu.sync_copy(s1_v.at[pl.ds(0, _NPB)], s1_hbm.at[pl.ds(node_base, _NPB)])
        pltpu.sync_copy(s2_v.at[pl.ds(0, _NPB)], s2_hbm.at[pl.ds(node_base, _NPB)])
        pltpu.sync_copy(mn_v.at[pl.ds(0, _NPB)], mn_hbm.at[pl.ds(node_base, _NPB)])
        pltpu.sync_copy(mx_v.at[pl.ds(0, _NPB)], mx_hbm.at[pl.ds(node_base, _NPB)])
        pltpu.sync_copy(cnt_v.at[pl.ds(0, _NPB)], cnt_hbm.at[pl.ds(node_base, _NPB)])


def _sc_stats_call(xj, edata_s, src_s, dst_s, starts):
    mesh = plsc.VectorSubcoreMesh(core_axis_name="c", subcore_axis_name="s",
                                  num_cores=_NC, num_subcores=_NS)
    f32 = jnp.float32
    out_type = [jax.ShapeDtypeStruct((_NPAD, 256), f32),
                jax.ShapeDtypeStruct((_NPAD, 256), f32),
                jax.ShapeDtypeStruct((_NPAD, 256), f32),
                jax.ShapeDtypeStruct((_NPAD, 256), f32),
                jax.ShapeDtypeStruct((_NPAD, 16), f32)]
    scratch = [pltpu.VMEM((_NB + 16,), jnp.int32),
               pltpu.VMEM((_K,), jnp.int32),
               pltpu.VMEM((_K,), jnp.int32),
               pltpu.VMEM((_K, 256), f32),
               pltpu.VMEM((_K, 256), f32),
               pltpu.VMEM((_NPB + 1, 256), f32),
               pltpu.VMEM((_NPB + 1, 256), f32),
               pltpu.VMEM((_NPB + 1, 256), f32),
               pltpu.VMEM((_NPB + 1, 256), f32),
               pltpu.VMEM((_NPB + 1, 16), f32),
               pltpu.SemaphoreType.DMA,
               pltpu.SemaphoreType.DMA]
    fn = pl.kernel(_sc_stats_body, out_type=out_type, mesh=mesh,
                   scratch_types=scratch,
                   compiler_params=pltpu.CompilerParams(
                       needs_layout_passes=False))
    return fn(xj, edata_s, src_s, dst_s, starts)


def _fold_layer(params, p):
    _ein = functools.partial(jnp.einsum, precision=lax.Precision.HIGHEST)
    pre_w = p['pre_w']
    A2 = pre_w[:, :_F, :].transpose(1, 0, 2).reshape(_F, _T * _F)
    B2 = pre_w[:, _F:2 * _F, :].transpose(1, 0, 2).reshape(_F, _T * _F)
    C2 = pre_w[:, 2 * _F:, :].transpose(1, 0, 2).reshape(_F, _T * _F)
    pre_b_flat = p['pre_b'].reshape(_T * _F)
    kmat = _ein('ab,bc,cd->ad', params['edge_emb_w'], p['edge_enc_w'], C2)
    cvec = _ein('b,bc->c', params['edge_emb_b'] @ p['edge_enc_w'] + p['edge_enc_b'], C2)
    w1 = jnp.concatenate([A2, B2], axis=1)
    b1 = jnp.concatenate([pre_b_flat, jnp.zeros((256,), jnp.float32)])
    lw = p['lin_w'].reshape(_T, _FO, _DIM)
    wx = _ein('tfo,tod->fd', p['post_w'][:, :_F, :], lw)

    def wblock(lo):
        blocks = []
        for s in range(4):
            ws = p['post_w'][:, lo + s * _F:lo + (s + 1) * _F, :]
            blocks.append(_ein('tgo,tod->tgd', ws, lw).reshape(_T * _F, _DIM))
        return jnp.concatenate(blocks, axis=0)

    wabc = jnp.concatenate([wblock(_F), wblock(5 * _F), wblock(9 * _F)], axis=1)
    bias = _ein('a,ab->b', p['post_b'].reshape(_T * _FO), p['lin_w']) + p['lin_b']
    kmat8 = jnp.concatenate([kmat, jnp.zeros((4, 256), jnp.float32)], axis=0)
    return dict(w1=w1, b1=b1[None, :], kmat8=kmat8, cvec=cvec[None, :],
                wx=wx, wabc=wabc, bias=bias[None, :],
                bn_g=p['bn_g'][None, :], bn_b=p['bn_b'][None, :])


def kernel(x, edge_index, edge_attr, batch, params):
    src = edge_index[0]
    dst = edge_index[1]
    key = dst // _NPB
    eiota = lax.iota(jnp.int32, _E)
    sk, perm = lax.sort([key, eiota], num_keys=1)
    starts = jnp.searchsorted(sk, jnp.arange(_NB + 1, dtype=jnp.int32)
                              ).astype(jnp.int32)
    starts = jnp.concatenate([starts, jnp.zeros((15,), jnp.int32)])
    src_s = jnp.concatenate([jnp.take(src, perm),
                             jnp.zeros((_EP - _E,), jnp.int32)])
    dst_s = jnp.concatenate([jnp.take(dst, perm),
                             jnp.zeros((_EP - _E,), jnp.int32)])
    ea_s = jnp.take(edge_attr, perm, axis=0)
    ea8 = jnp.concatenate(
        [jnp.concatenate([ea_s, jnp.zeros((_E, 4), jnp.float32)], axis=1),
         jnp.zeros((_EP - _E, 8), jnp.float32)], axis=0)
    batch2d = batch[:, None]

    folds = [_fold_layer(params, p) for p in params['convs']]

    f = folds[0]
    h, xi, xj = _embpre_call(x, params['node_emb_w'],
                             params['node_emb_b'][None, :], f['w1'], f['b1'])
    out = None
    bn = None
    for li in range(4):
        f = folds[li]
        if li > 0:
            h, xi, xj = _prebn_call(out, bn, folds[li - 1]['bn_g'],
                                    folds[li - 1]['bn_b'], f['w1'], f['b1'])
        edata = _edata_call(ea8, f['kmat8'], f['cvec'])
        s1, s2, mn, mx, cnt = _sc_stats_call(xj, edata, src_s, dst_s, starts)
        out, bn = _post_call(s1[:_N], s2[:_N], mn[:_N], mx[:_N], cnt[:_N],
                             xi, h, f['wabc'], f['wx'], f['bias'])
    f = folds[3]
    per_atom, g = _final_call(out, bn, f['bn_g'], f['bn_b'], batch2d,
                              params['mlp_w1'], params['mlp_b1'][None, :],
                              params['mlp_w2'], params['mlp_b2'][None, :],
                              params['mlp_w3'], params['mlp_b3'][None, :])
    return (g.reshape(-1), per_atom)


# speedup vs baseline: 68.7551x; 1.2309x over previous
import functools

import jax
import jax.numpy as jnp
import numpy as np
from jax import lax
from jax.experimental import pallas as pl
from jax.experimental.pallas import tpu as pltpu
from jax.experimental.pallas import tpu_sc as plsc

_DEG = [1, 72, 201, 816, 1790, 3756, 6923, 12768, 20286, 31710, 51623,
        82296, 124280, 177576, 251115, 326064, 395760, 456840, 506179,
        516200, 507003, 493746, 489256, 453936, 420025, 411320, 427761,
        420700, 420500, 426780, 414284, 407008, 394053, 360910, 322245,
        313704, 282902, 270940, 237783, 209000, 193766, 177870, 162110,
        144848, 121230, 112700, 93483, 88512, 72275, 80700, 68799, 56784,
        42665, 30996, 25630, 12936, 9804, 8584, 5251, 3480, 3111, 2728,
        1890, 1472, 1235, 330, 201, 68, 69, 0, 71]
_dega = np.asarray(_DEG, dtype=np.float64)
_bins = np.arange(len(_DEG), dtype=np.float64)
_AVG_DEG_LOG = float((np.log(_bins + 1.0) * _dega).sum() / _dega.sum())

_N = 10000
_E = 160000
_DIM = 64
_T = 4
_F = 64
_FO = 16
_NG = 256
_EPS_BN = 1e-5

_NB = 128
_NPB = 80
_NPAD = _NB * _NPB
_K = 32
_EP = 160256
_BIG = 3.0e38


def _mm(a, b):
    return jnp.dot(a, b, preferred_element_type=jnp.float32,
                   precision=lax.Precision.HIGHEST)


def _embpre_body(x_ref, we_ref, be_ref, w1_ref, b1_ref, h_ref, xi_ref, xj_ref):
    h = _mm(x_ref[...], we_ref[...]) + be_ref[...]
    h_ref[...] = h
    xixj = _mm(h, w1_ref[...]) + b1_ref[...]
    xi_ref[...] = xixj[:, :256]
    xj_ref[...] = xixj[:, 256:]


def _prebn_body(o_ref, bn_ref, g_ref, b_ref, w1_ref, b1_ref,
                h_ref, xi_ref, xj_ref):
    s = bn_ref[...]
    m = s[0:1, :] * (1.0 / _N)
    var = s[1:2, :] * (1.0 / _N) - m * m
    o = o_ref[...]
    h = jax.nn.relu(g_ref[...] * (o - m) * lax.rsqrt(var + _EPS_BN) + b_ref[...])
    h_ref[...] = h
    xixj = _mm(h, w1_ref[...]) + b1_ref[...]
    xi_ref[...] = xixj[:, :256]
    xj_ref[...] = xixj[:, 256:]


def _edata_body(ea_ref, k_ref, c_ref, out_ref):
    out_ref[...] = _mm(ea_ref[...], k_ref[...]) + c_ref[...]


def _post_body(s1_ref, s2_ref, mn_ref, mx_ref, cnt_ref, xi_ref, h_ref,
               wabc_ref, wx_ref, bias_ref, out_ref, bn_ref):
    i = pl.program_id(0)
    c = cnt_ref[...][:, 0:1]
    has = c > 0.0
    cc = jnp.maximum(c, 1.0)
    inv = 1.0 / cc
    xi = xi_ref[...]
    s1 = s1_ref[...]
    e1 = s1 * inv
    mean = jnp.where(has, xi + e1, 0.0)
    mn = jnp.where(has, xi + mn_ref[...], 0.0)
    mx = jnp.where(has, xi + mx_ref[...], 0.0)
    var = s2_ref[...] * inv - e1 * e1
    std = jnp.sqrt(jax.nn.relu(var) + 1e-5)
    stat = jnp.concatenate([mean, mn, mx, std], axis=1)
    u = _mm(stat, wabc_ref[...])
    alpha = jnp.log(cc + 1.0) * (1.0 / _AVG_DEG_LOG)
    beta = _AVG_DEG_LOG / jnp.log(cc + 1.0)
    out = (_mm(h_ref[...], wx_ref[...]) + u[:, :64] + alpha * u[:, 64:128]
           + beta * u[:, 128:] + bias_ref[...])
    out_ref[...] = out

    @pl.when(i == 0)
    def _():
        bn_ref[...] = jnp.zeros_like(bn_ref)

    part = jnp.concatenate([jnp.sum(out, axis=0, keepdims=True),
                            jnp.sum(out * out, axis=0, keepdims=True)], axis=0)
    bn_ref[...] += part


def _final_body(o_ref, bn_ref, g_ref, b_ref, batch_ref, w1_ref, b1_ref,
                w2_ref, b2_ref, w3_ref, b3_ref, h_ref, gout_ref):
    s = bn_ref[...]
    m = s[0:1, :] * (1.0 / _N)
    var = s[1:2, :] * (1.0 / _N) - m * m
    h = jax.nn.relu(g_ref[...] * (o_ref[...] - m) * lax.rsqrt(var + _EPS_BN)
                    + b_ref[...])
    h_ref[...] = h
    onehot = (batch_ref[...] == lax.broadcasted_iota(jnp.int32, (1, _NG), 1)
              ).astype(jnp.float32)
    g = lax.dot_general(onehot, h, (((0,), (0,)), ((), ())),
                        preferred_element_type=jnp.float32,
                        precision=lax.Precision.HIGHEST)
    g = jax.nn.relu(_mm(g, w1_ref[...]) + b1_ref[...])
    g = jax.nn.relu(_mm(g, w2_ref[...]) + b2_ref[...])
    gout_ref[...] = _mm(g, w3_ref[...]) + b3_ref[...]


_ROWB = 2000


def _full(shape):
    return pl.BlockSpec(shape, lambda *a: tuple(0 for _ in shape))


def _rows(cols):
    return pl.BlockSpec((_ROWB, cols), lambda i: (i, 0))


def _embpre_call(x, we, be, w1, b1):
    return pl.pallas_call(
        _embpre_body,
        grid=(_N // _ROWB,),
        in_specs=[_rows(14), _full((14, 64)), _full((1, 64)),
                  _full((64, 512)), _full((1, 512))],
        out_specs=[_rows(64), _rows(256), _rows(256)],
        out_shape=[jax.ShapeDtypeStruct((_N, 64), jnp.float32),
                   jax.ShapeDtypeStruct((_N, 256), jnp.float32),
                   jax.ShapeDtypeStruct((_N, 256), jnp.float32)],
    )(x, we, be, w1, b1)


def _prebn_call(o, bn, g, b, w1, b1):
    return pl.pallas_call(
        _prebn_body,
        grid=(_N // _ROWB,),
        in_specs=[_rows(64), _full((2, 64)), _full((1, 64)), _full((1, 64)),
                  _full((64, 512)), _full((1, 512))],
        out_specs=[_rows(64), _rows(256), _rows(256)],
        out_shape=[jax.ShapeDtypeStruct((_N, 64), jnp.float32),
                   jax.ShapeDtypeStruct((_N, 256), jnp.float32),
                   jax.ShapeDtypeStruct((_N, 256), jnp.float32)],
    )(o, bn, g, b, w1, b1)


def _edata_call(ea8, kmat, cvec):
    return pl.pallas_call(
        _edata_body,
        grid=(_EP // 2504,),
        in_specs=[pl.BlockSpec((2504, 8), lambda i: (i, 0)),
                  _full((8, 256)), _full((1, 256))],
        out_specs=pl.BlockSpec((2504, 256), lambda i: (i, 0)),
        out_shape=jax.ShapeDtypeStruct((_EP, 256), jnp.float32),
    )(ea8, kmat, cvec)


def _post_call(s1, s2, mn, mx, cnt, xi, h, wabc, wx, bias):
    return pl.pallas_call(
        _post_body,
        grid=(_N // _ROWB,),
        in_specs=[_rows(256), _rows(256), _rows(256), _rows(256),
                  _rows(16), _rows(256), _rows(64),
                  _full((1024, 192)), _full((64, 64)), _full((1, 64))],
        out_specs=[_rows(64), _full((2, 64))],
        out_shape=[jax.ShapeDtypeStruct((_N, 64), jnp.float32),
                   jax.ShapeDtypeStruct((2, 64), jnp.float32)],
    )(s1, s2, mn, mx, cnt, xi, h, wabc, wx, bias)


def _final_call(o, bn, g, b, batch2d, w1, b1, w2, b2, w3, b3):
    return pl.pallas_call(
        _final_body,
        in_specs=[_full((_N, 64)), _full((2, 64)), _full((1, 64)),
                  _full((1, 64)), _full((_N, 1)), _full((64, 50)),
                  _full((1, 50)), _full((50, 25)), _full((1, 25)),
                  _full((25, 1)), _full((1, 1))],
        out_specs=[_full((_N, 64)), _full((_NG, 1))],
        out_shape=[jax.ShapeDtypeStruct((_N, 64), jnp.float32),
                   jax.ShapeDtypeStruct((_NG, 1), jnp.float32)],
    )(o, bn, g, b, batch2d, w1, b1, w2, b2, w3, b3)


_NC = 2
_NS = 16
_NW = _NC * _NS
_BPW = _NB // _NW


def _sc_read_scalar(vec_ref, idx):
    lanes = jnp.full((16,), idx, dtype=jnp.int32)
    vals = plsc.load_gather(vec_ref, [lanes])
    return jnp.max(vals, axis=0)


def _extract_lane_i32(vec, j):
    lane = lax.iota(jnp.int32, 16)
    return jnp.max(jnp.where(lane == j, vec, -1), axis=0)


def _sc_stats_body(xj_hbm, ed_hbm, src_hbm, dst_hbm, starts_hbm,
                   s1_hbm, s2_hbm, mn_hbm, mx_hbm, cnt_hbm,
                   starts_v, srcA, dstA, edA, xjA, srcB, dstB, edB, xjB,
                   s1_v, s2_v, mn_v, mx_v, cnt_v,
                   semA_sd, semA_ed, semA_xj, semB_sd, semB_ed, semB_xj):
    wid = lax.axis_index("s") * _NC + lax.axis_index("c")
    pltpu.sync_copy(starts_hbm, starts_v)
    iota = lax.iota(jnp.int32, 16)
    zf = jnp.zeros((16,), jnp.float32)
    big = jnp.full((16,), _BIG, jnp.float32)
    onehot0 = jnp.where(iota == 0, 1.0, 0.0).astype(jnp.float32)
    for k in range(_BPW):
        b = wid * _BPW + k
        start = _sc_read_scalar(starts_v, b)
        end = _sc_read_scalar(starts_v, b + 1)
        node_base = b * _NPB
        astart = (start // 8) * 8
        nch = (end - astart + _K - 1) // _K
        npairs = (nch + 3) // 2

        def issue_sd(c, sv, dv, sem):
            bb = astart + c * _K
            pltpu.async_copy(src_hbm.at[pl.ds(bb, _K)], sv, sem)
            pltpu.async_copy(dst_hbm.at[pl.ds(bb, _K)], dv, sem)

        def wait_sd(sv, dv, sem):
            pltpu.make_async_copy(src_hbm.at[pl.ds(0, _K)], sv, sem).wait()
            pltpu.make_async_copy(dst_hbm.at[pl.ds(0, _K)], dv, sem).wait()

        def issue_gathers(c, sv, ev, xv, sem_e, sem_x):
            bb = astart + c * _K
            pltpu.async_copy(ed_hbm.at[pl.ds(bb, _K)], ev, sem_e)
            pltpu.async_copy(xj_hbm.at[sv], xv, sem_x)

        def wait_gathers(sv, ev, xv, sem_e, sem_x):
            pltpu.make_async_copy(ed_hbm.at[pl.ds(0, _K)], ev, sem_e).wait()
            pltpu.make_async_copy(xj_hbm.at[sv], xv, sem_x).wait()

        issue_sd(0, srcA, dstA, semA_sd)
        issue_sd(1, srcB, dstB, semB_sd)

        def zrow(i, _):
            for c in range(16):
                s1_v[pl.ds(i * 256 + c * 16, 16)] = zf
                s2_v[pl.ds(i * 256 + c * 16, 16)] = zf
                mn_v[pl.ds(i * 256 + c * 16, 16)] = big
                mx_v[pl.ds(i * 256 + c * 16, 16)] = -big
            cnt_v[pl.ds(i * 16, 16)] = zf
            return 0

        lax.fori_loop(0, _NPB + 1, zrow, 0)

        wait_sd(srcA, dstA, semA_sd)
        issue_gathers(0, srcA, edA, xjA, semA_ed, semA_xj)

        def compute(c, dvs, ev, xv):
            for g in range(_K // 16):
                gidx = astart + c * _K + g * 16 + iota
                valid = (gidx >= start) & (gidx < end)
                dstloc = jnp.where(valid, dvs[g] - node_base, _NPB)

                def edge_body(j, _):
                    row = _extract_lane_i32(dstloc, j)
                    off = row * 256
                    er = g * 16 + j
                    plsc.addupdate(cnt_v.at[pl.ds(row * 16, 16)], onehot0)
                    for cc in range(16):
                        xc = xv[er, pl.ds(cc * 16, 16)]
                        ec = ev[er, pl.ds(cc * 16, 16)]
                        y = xc + ec
                        plsc.addupdate(s1_v.at[pl.ds(off + cc * 16, 16)], y)
                        plsc.addupdate(s2_v.at[pl.ds(off + cc * 16, 16)], y * y)
                        cur = mn_v[pl.ds(off + cc * 16, 16)]
                        mn_v[pl.ds(off + cc * 16, 16)] = jnp.minimum(cur, y)
                        cur = mx_v[pl.ds(off + cc * 16, 16)]
                        mx_v[pl.ds(off + cc * 16, 16)] = jnp.maximum(cur, y)
                    return 0

                lax.fori_loop(0, 16, edge_body, 0)

        def phase(c, sv, dv, ev, xv, sem_sd, sem_ed, sem_xj,
                  nsv, ndv, nev, nxv, nsem_sd, nsem_ed, nsem_xj):
            wait_gathers(sv, ev, xv, sem_ed, sem_xj)
            wait_sd(nsv, ndv, nsem_sd)
            issue_gathers(c + 1, nsv, nev, nxv, nsem_ed, nsem_xj)
            dvs = [dv[pl.ds(0, 16)], dv[pl.ds(16, 16)]]
            issue_sd(c + 2, sv, dv, sem_sd)
            compute(c, dvs, ev, xv)

        def pair_body(p, _):
            c = 2 * p
            phase(c, srcA, dstA, edA, xjA, semA_sd, semA_ed, semA_xj,
                  srcB, dstB, edB, xjB, semB_sd, semB_ed, semB_xj)
            phase(c + 1, srcB, dstB, edB, xjB, semB_sd, semB_ed, semB_xj,
                  srcA, dstA, edA, xjA, semA_sd, semA_ed, semA_xj)
            return 0

        lax.fori_loop(0, npairs, pair_body, 0)
        wait_gathers(srcA, edA, xjA, semA_ed, semA_xj)
        wait_sd(srcB, dstB, semB_sd)

        pltpu.sync_copy(s1_v.at[pl.ds(0, _NPB * 256)],
                        s1_hbm.at[pl.ds(node_base * 256, _NPB * 256)])
        pltpu.sync_copy(s2_v.at[pl.ds(0, _NPB * 256)],
                        s2_hbm.at[pl.ds(node_base * 256, _NPB * 256)])
        pltpu.sync_copy(mn_v.at[pl.ds(0, _NPB * 256)],
                        mn_hbm.at[pl.ds(node_base * 256, _NPB * 256)])
        pltpu.sync_copy(mx_v.at[pl.ds(0, _NPB * 256)],
                        mx_hbm.at[pl.ds(node_base * 256, _NPB * 256)])
        pltpu.sync_copy(cnt_v.at[pl.ds(0, _NPB * 16)],
                        cnt_hbm.at[pl.ds(node_base * 16, _NPB * 16)])


def _sc_stats_call(xj, edata_s, src_s, dst_s, starts):
    mesh = plsc.VectorSubcoreMesh(core_axis_name="c", subcore_axis_name="s",
                                  num_cores=_NC, num_subcores=_NS)
    f32 = jnp.float32
    out_type = [jax.ShapeDtypeStruct((_NPAD * 256,), f32),
                jax.ShapeDtypeStruct((_NPAD * 256,), f32),
                jax.ShapeDtypeStruct((_NPAD * 256,), f32),
                jax.ShapeDtypeStruct((_NPAD * 256,), f32),
                jax.ShapeDtypeStruct((_NPAD * 16,), f32)]
    scratch = ([pltpu.VMEM((_NB + 16,), jnp.int32)]
               + [pltpu.VMEM((_K,), jnp.int32),
                  pltpu.VMEM((_K,), jnp.int32),
                  pltpu.VMEM((_K, 256), f32),
                  pltpu.VMEM((_K, 256), f32)] * 2
               + [pltpu.VMEM(((_NPB + 1) * 256,), f32)] * 4
               + [pltpu.VMEM(((_NPB + 1) * 16,), f32)]
               + [pltpu.SemaphoreType.DMA] * 6)
    fn = pl.kernel(_sc_stats_body, out_type=out_type, mesh=mesh,
                   scratch_types=scratch,
                   compiler_params=pltpu.CompilerParams(
                       needs_layout_passes=False))
    return fn(xj, edata_s, src_s, dst_s, starts)


def _fold_layer(params, p):
    _ein = functools.partial(jnp.einsum, precision=lax.Precision.HIGHEST)
    pre_w = p['pre_w']
    A2 = pre_w[:, :_F, :].transpose(1, 0, 2).reshape(_F, _T * _F)
    B2 = pre_w[:, _F:2 * _F, :].transpose(1, 0, 2).reshape(_F, _T * _F)
    C2 = pre_w[:, 2 * _F:, :].transpose(1, 0, 2).reshape(_F, _T * _F)
    pre_b_flat = p['pre_b'].reshape(_T * _F)
    kmat = _ein('ab,bc,cd->ad', params['edge_emb_w'], p['edge_enc_w'], C2)
    cvec = _ein('b,bc->c', params['edge_emb_b'] @ p['edge_enc_w'] + p['edge_enc_b'], C2)
    w1 = jnp.concatenate([A2, B2], axis=1)
    b1 = jnp.concatenate([pre_b_flat, jnp.zeros((256,), jnp.float32)])
    lw = p['lin_w'].reshape(_T, _FO, _DIM)
    wx = _ein('tfo,tod->fd', p['post_w'][:, :_F, :], lw)

    def wblock(lo):
        blocks = []
        for s in range(4):
            ws = p['post_w'][:, lo + s * _F:lo + (s + 1) * _F, :]
            blocks.append(_ein('tgo,tod->tgd', ws, lw).reshape(_T * _F, _DIM))
        return jnp.concatenate(blocks, axis=0)

    wabc = jnp.concatenate([wblock(_F), wblock(5 * _F), wblock(9 * _F)], axis=1)
    bias = _ein('a,ab->b', p['post_b'].reshape(_T * _FO), p['lin_w']) + p['lin_b']
    kmat8 = jnp.concatenate([kmat, jnp.zeros((4, 256), jnp.float32)], axis=0)
    return dict(w1=w1, b1=b1[None, :], kmat8=kmat8, cvec=cvec[None, :],
                wx=wx, wabc=wabc, bias=bias[None, :],
                bn_g=p['bn_g'][None, :], bn_b=p['bn_b'][None, :])


def kernel(x, edge_index, edge_attr, batch, params):
    src = edge_index[0]
    dst = edge_index[1]
    key = dst // _NPB
    eiota = lax.iota(jnp.int32, _E)
    sk, perm = lax.sort([key, eiota], num_keys=1)
    starts = jnp.searchsorted(sk, jnp.arange(_NB + 1, dtype=jnp.int32)
                              ).astype(jnp.int32)
    starts = jnp.concatenate([starts, jnp.zeros((15,), jnp.int32)])
    src_s = jnp.concatenate([jnp.take(src, perm),
                             jnp.zeros((_EP - _E,), jnp.int32)])
    dst_s = jnp.concatenate([jnp.take(dst, perm),
                             jnp.zeros((_EP - _E,), jnp.int32)])
    ea_s = jnp.take(edge_attr, perm, axis=0)
    ea8 = jnp.concatenate(
        [jnp.concatenate([ea_s, jnp.zeros((_E, 4), jnp.float32)], axis=1),
         jnp.zeros((_EP - _E, 8), jnp.float32)], axis=0)
    batch2d = batch[:, None]

    folds = [_fold_layer(params, p) for p in params['convs']]

    f = folds[0]
    h, xi, xj = _embpre_call(x, params['node_emb_w'],
                             params['node_emb_b'][None, :], f['w1'], f['b1'])
    out = None
    bn = None
    for li in range(4):
        f = folds[li]
        if li > 0:
            h, xi, xj = _prebn_call(out, bn, folds[li - 1]['bn_g'],
                                    folds[li - 1]['bn_b'], f['w1'], f['b1'])
        edata = _edata_call(ea8, f['kmat8'], f['cvec'])
        s1, s2, mn, mx, cnt = _sc_stats_call(xj, edata, src_s, dst_s, starts)
        s1 = s1.reshape(_NPAD, 256)[:_N]
        s2 = s2.reshape(_NPAD, 256)[:_N]
        mn = mn.reshape(_NPAD, 256)[:_N]
        mx = mx.reshape(_NPAD, 256)[:_N]
        cnt = cnt.reshape(_NPAD, 16)[:_N]
        out, bn = _post_call(s1, s2, mn, mx, cnt,
                             xi, h, f['wabc'], f['wx'], f['bias'])
    f = folds[3]
    per_atom, g = _final_call(out, bn, f['bn_g'], f['bn_b'], batch2d,
                              params['mlp_w1'], params['mlp_b1'][None, :],
                              params['mlp_w2'], params['mlp_b2'][None, :],
                              params['mlp_w3'], params['mlp_b3'][None, :])
    return (g.reshape(-1), per_atom)
